# baseline (speedup 1.0000x reference)
import jax
import jax.numpy as jnp
from jax import lax
from jax.experimental import pallas as pl
from jax.experimental.pallas import tpu as pltpu

_EPS = 1e-06
_BPG = 2


def _body(xt_ref, out_ref, qt0_ref, rate_ref, o_ref):
    lt, s = out_ref.shape[1], out_ref.shape[2]
    f32, bf16 = jnp.float32, jnp.bfloat16
    bi = pl.program_id(0)
    iota = lax.broadcasted_iota(jnp.int32, (lt, s), 1)
    diag = lax.broadcasted_iota(jnp.int32, (s, s), 0) == \
        lax.broadcasted_iota(jnp.int32, (s, s), 1)
    dn_t = (((1,), (1,)), ((), ()))
    for j in range(_BPG):
        jb = bi * _BPG + j
        xt = xt_ref[jb, 0, :]
        qt0 = qt0_ref[jb]
        oh16 = (xt[:, None] == iota).astype(bf16)
        recip = (1.0 / (qt0 + _EPS)).astype(bf16)
        ratez = jnp.where(diag, 0.0, rate_ref[jb]).astype(bf16)
        recip_g = lax.dot_general(oh16, recip, dn_t, preferred_element_type=f32)
        fwd_g = lax.dot_general(oh16, ratez, dn_t, preferred_element_type=f32)
        d16 = (out_ref[j] * recip_g).astype(bf16)
        score = jnp.dot(d16, qt0.astype(bf16), preferred_element_type=f32)
        o_ref[j] = fwd_g * score


def kernel(output, xt, t, qt0, rate):
    del t
    b, l, s = output.shape
    xt3 = xt.reshape(b, 1, l)
    return pl.pallas_call(
        _body,
        grid=(b // _BPG,),
        in_specs=[
            pl.BlockSpec((b, 1, l), lambda bi: (0, 0, 0)),
            pl.BlockSpec((_BPG, l, s), lambda bi: (bi, 0, 0)),
            pl.BlockSpec((b, s, s), lambda bi: (0, 0, 0)),
            pl.BlockSpec((b, s, s), lambda bi: (0, 0, 0)),
        ],
        out_specs=pl.BlockSpec((_BPG, l, s), lambda bi: (bi, 0, 0)),
        out_shape=jax.ShapeDtypeStruct((b, l, s), jnp.float32),
        compiler_params=pltpu.CompilerParams(
            dimension_semantics=("arbitrary",)),
    )(xt3, output, qt0, rate)


# speedup vs baseline: 1.0060x; 1.0060x over previous
import jax
import jax.numpy as jnp
from jax import lax
from jax.experimental import pallas as pl
from jax.experimental.pallas import tpu as pltpu

_EPS = 1e-06
_BPG = 2


def _body(xt_ref, out_ref, qt0_ref, rate_ref, o_ref):
    lt, s = out_ref.shape[1], out_ref.shape[2]
    f32, bf16 = jnp.float32, jnp.bfloat16
    iota = lax.broadcasted_iota(jnp.int32, (lt, s), 1)
    diag = lax.broadcasted_iota(jnp.int32, (s, s), 0) == \
        lax.broadcasted_iota(jnp.int32, (s, s), 1)
    dn_t = (((1,), (1,)), ((), ()))
    for j in range(_BPG):
        xt = xt_ref[j, 0, :]
        oh16 = (xt[:, None] == iota).astype(bf16)
        recip = (1.0 / (qt0_ref[j] + _EPS)).astype(bf16)
        ratez = jnp.where(diag, 0.0, rate_ref[j]).astype(bf16)
        recip_g = lax.dot_general(oh16, recip, dn_t, preferred_element_type=f32)
        fwd_g = lax.dot_general(oh16, ratez, dn_t, preferred_element_type=f32)
        d16 = (out_ref[j] * recip_g).astype(bf16)
        score = jnp.dot(d16, qt0_ref[j].astype(bf16), preferred_element_type=f32)
        o_ref[j] = fwd_g * score


def kernel(output, xt, t, qt0, rate):
    del t
    b, l, s = output.shape
    xt3 = xt.reshape(b, 1, l)
    return pl.pallas_call(
        _body,
        grid=(b // _BPG,),
        in_specs=[
            pl.BlockSpec((_BPG, 1, l), lambda bi: (bi, 0, 0)),
            pl.BlockSpec((_BPG, l, s), lambda bi: (bi, 0, 0)),
            pl.BlockSpec((_BPG, s, s), lambda bi: (bi, 0, 0)),
            pl.BlockSpec((_BPG, s, s), lambda bi: (bi, 0, 0)),
        ],
        out_specs=pl.BlockSpec((_BPG, l, s), lambda bi: (bi, 0, 0)),
        out_shape=jax.ShapeDtypeStruct((b, l, s), jnp.float32),
        compiler_params=pltpu.CompilerParams(
            dimension_semantics=("parallel",)),
    )(xt3, output, qt0, rate)
